# baseline (speedup 1.0000x reference)
import functools

import jax
import jax.numpy as jnp
from jax import lax
from jax.experimental import pallas as pl
from jax.experimental.pallas import tpu as pltpu
from jax.experimental.pallas import tpu_sc as plsc

N_BALL = 50000
N_CTX = 10000
D = 128
H = 4
C = D // H

_NC = 2
_NS = 16
_NW = _NC * _NS

_SG_BLK = 128
_SG_NBLK = 25
_SG_TROW = 32
_SG_EPAD = _NW * _SG_NBLK * _SG_BLK
_SG_COLS = 128
_SG_ROWS = 10016
_SG_NCHUNK = _SG_ROWS // 32
_SG_CPT = (_SG_NCHUNK + _NS - 1) // _NS


def _sage_sc_body(x_hbm, sidx_hbm, didx_hbm, out_hbm,
                  acc, sidx_v, didx_v, rows_v, sem):
    c = lax.axis_index("c")
    s = lax.axis_index("s")
    tile = c * _NS + s

    zrow = jnp.zeros((16,), jnp.float32)

    def zrow_body(j, _):
        for l in range(8):
            rows_v[j, pl.ds(l * 16, 16)] = zrow
        return 0

    lax.fori_loop(0, _SG_BLK, zrow_body, 0)

    for r in range(_SG_CPT):
        chunk = s * _SG_CPT + r

        @pl.when(chunk < _SG_NCHUNK)
        def _zero_chunk(chunk=chunk):
            pltpu.sync_copy(rows_v.at[pl.ds(0, 32)],
                            acc.at[pl.ds(chunk * 32, 32)])

    plsc.subcore_barrier()

    pltpu.sync_copy(sidx_hbm.at[pl.ds(tile * _SG_TROW, _SG_TROW)], sidx_v)
    pltpu.sync_copy(didx_hbm.at[pl.ds(tile * _SG_TROW, _SG_TROW)], didx_v)

    def blk_body(j, _):
        pltpu.async_copy(x_hbm.at[sidx_v.at[j]], rows_v, sem).wait()
        pltpu.sync_copy(rows_v, acc.at[didx_v.at[j]], add=True)
        return 0

    lax.fori_loop(0, _SG_NBLK, blk_body, 0)
    plsc.subcore_barrier()

    for r in range(_SG_CPT):
        chunk = s * _SG_CPT + r

        @pl.when(chunk < _SG_NCHUNK)
        def _flush_chunk(chunk=chunk):
            pltpu.sync_copy(acc.at[pl.ds(chunk * 32, 32)],
                            rows_v.at[pl.ds(0, 32)])
            pltpu.sync_copy(rows_v.at[pl.ds(0, 32)],
                            out_hbm.at[pl.ds(c * _SG_ROWS + chunk * 32, 32)])


def _sage_sc(xaug, s3, d3):
    e = s3.shape[0]
    pad = _SG_EPAD - e
    ar = lax.iota(jnp.int32, pad)
    trash = _SG_ROWS - N_CTX
    s3p = jnp.concatenate([s3, (ar * 64) % N_BALL]
                          ).reshape(_NW, _SG_NBLK, _SG_BLK)
    d3p = jnp.concatenate([d3, N_CTX + (ar % trash)]
                          ).reshape(_NW, _SG_NBLK, _SG_BLK)
    zpad = jnp.zeros((_NW, _SG_TROW - _SG_NBLK, _SG_BLK), jnp.int32)
    s3p = jnp.concatenate([s3p, zpad], axis=1).reshape(-1, _SG_BLK)
    d3p = jnp.concatenate([d3p, zpad], axis=1).reshape(-1, _SG_BLK)
    kfn = pl.kernel(
        _sage_sc_body,
        out_type=jax.ShapeDtypeStruct((_NC * _SG_ROWS, _SG_COLS),
                                      jnp.float32),
        mesh=plsc.VectorSubcoreMesh(core_axis_name="c", subcore_axis_name="s"),
        scratch_types=[
            pltpu.VMEM_SHARED((_SG_ROWS, _SG_COLS), jnp.float32),
            pltpu.VMEM((_SG_TROW, _SG_BLK), jnp.int32),
            pltpu.VMEM((_SG_TROW, _SG_BLK), jnp.int32),
            pltpu.VMEM((_SG_BLK, _SG_COLS), jnp.float32),
            pltpu.SemaphoreType.DMA,
        ],
    )
    out = kfn(xaug, s3p, d3p)
    feat = out[:N_CTX] + out[_SG_ROWS:_SG_ROWS + N_CTX]
    return feat


def _pre_kernel(x_ref, w_ref, b_ref, o_ref):
    o_ref[...] = (
        jnp.dot(x_ref[...], w_ref[...], preferred_element_type=jnp.float32)
        + b_ref[...]
    )


def _fused_matmul(x, w, b, block):
    n, kdim = x.shape
    m = w.shape[1]
    grid = n // block
    return pl.pallas_call(
        _pre_kernel,
        grid=(grid,),
        in_specs=[
            pl.BlockSpec((block, kdim), lambda i: (i, 0)),
            pl.BlockSpec((kdim, m), lambda i: (0, 0)),
            pl.BlockSpec((1, m), lambda i: (0, 0)),
        ],
        out_specs=pl.BlockSpec((block, m), lambda i: (i, 0)),
        out_shape=jax.ShapeDtypeStruct((n, m), jnp.float32),
    )(x, w, b)


def _post_ball_kernel(numr_ref, rr_ref, nump_ref, rp_ref, skip_ref, x_ref,
                      gb_ref, eh_ref, lnw_ref, lnb_ref, o_ref):
    eh = eh_ref[...]
    t = (numr_ref[...] * jnp.dot(rr_ref[...], eh, preferred_element_type=jnp.float32)
         + nump_ref[...] * jnp.dot(rp_ref[...], eh, preferred_element_type=jnp.float32)
         + skip_ref[...] + x_ref[...] + gb_ref[...])
    mu = jnp.mean(t, axis=-1, keepdims=True)
    d = t - mu
    var = jnp.mean(d * d, axis=-1, keepdims=True)
    o_ref[...] = d * jax.lax.rsqrt(var + 1e-5) * lnw_ref[...] + lnb_ref[...]


def _post_ball(num_rel, rec_rel, num_prec, rec_prec, skip, x, gat_bias, lnw, lnb,
               block):
    n = x.shape[0]
    eh = jnp.repeat(jnp.eye(H, dtype=jnp.float32), C, axis=1)
    grid = n // block
    return pl.pallas_call(
        _post_ball_kernel,
        grid=(grid,),
        in_specs=[
            pl.BlockSpec((block, D), lambda i: (i, 0)),
            pl.BlockSpec((block, H), lambda i: (i, 0)),
            pl.BlockSpec((block, D), lambda i: (i, 0)),
            pl.BlockSpec((block, H), lambda i: (i, 0)),
            pl.BlockSpec((block, D), lambda i: (i, 0)),
            pl.BlockSpec((block, D), lambda i: (i, 0)),
            pl.BlockSpec((1, D), lambda i: (0, 0)),
            pl.BlockSpec((H, D), lambda i: (0, 0)),
            pl.BlockSpec((1, D), lambda i: (0, 0)),
            pl.BlockSpec((1, D), lambda i: (0, 0)),
        ],
        out_specs=pl.BlockSpec((block, D), lambda i: (i, 0)),
        out_shape=jax.ShapeDtypeStruct((n, D), jnp.float32),
    )(num_rel, rec_rel, num_prec, rec_prec, skip, x,
      gat_bias.reshape(1, D), eh, lnw.reshape(1, D), lnb.reshape(1, D))


def _post_ctx_kernel(mean_ref, x_ref, wl_ref, wr_ref, bl_ref, lnw_ref, lnb_ref,
                     o_ref):
    t = (jnp.dot(mean_ref[...], wl_ref[...], preferred_element_type=jnp.float32)
         + jnp.dot(x_ref[...], wr_ref[...], preferred_element_type=jnp.float32)
         + bl_ref[...] + x_ref[...])
    mu = jnp.mean(t, axis=-1, keepdims=True)
    d = t - mu
    var = jnp.mean(d * d, axis=-1, keepdims=True)
    o_ref[...] = d * jax.lax.rsqrt(var + 1e-5) * lnw_ref[...] + lnb_ref[...]


def _post_ctx(mean, x_ctx, wl, wr, bl, lnw, lnb, block):
    n = x_ctx.shape[0]
    grid = n // block
    return pl.pallas_call(
        _post_ctx_kernel,
        grid=(grid,),
        in_specs=[
            pl.BlockSpec((block, D), lambda i: (i, 0)),
            pl.BlockSpec((block, D), lambda i: (i, 0)),
            pl.BlockSpec((D, D), lambda i: (0, 0)),
            pl.BlockSpec((D, D), lambda i: (0, 0)),
            pl.BlockSpec((1, D), lambda i: (0, 0)),
            pl.BlockSpec((1, D), lambda i: (0, 0)),
            pl.BlockSpec((1, D), lambda i: (0, 0)),
        ],
        out_specs=pl.BlockSpec((block, D), lambda i: (i, 0)),
        out_shape=jax.ShapeDtypeStruct((n, D), jnp.float32),
    )(mean, x_ctx, wl, wr, bl.reshape(1, D), lnw.reshape(1, D),
      lnb.reshape(1, D))


def kernel(x_ball, x_ctx, edge_attr_prec, gat_wl, gat_bl, gat_wr, gat_br,
           gat_att, gat_bias, tr_wq, tr_bq, tr_wk, tr_bk, tr_wv, tr_bv, tr_we,
           tr_wskip, tr_bskip, sage_wl, sage_bl, sage_wr, ln_ball_w, ln_ball_b,
           ln_ctx_w, ln_ctx_b, edge_index_rel, edge_index_prec, edge_index_inf):
    w6 = jnp.concatenate([gat_wl, gat_wr, tr_wq, tr_wk, tr_wv, tr_wskip], axis=1)
    b6 = jnp.concatenate([gat_bl, gat_br, tr_bq, tr_bk, tr_bv, tr_bskip])
    pre = _fused_matmul(x_ball, w6, b6.reshape(1, 6 * D), 2000)
    xl = pre[:, 0 * D:1 * D]
    xr = pre[:, 1 * D:2 * D]
    q = pre[:, 2 * D:3 * D]
    k = pre[:, 3 * D:4 * D]
    v = pre[:, 4 * D:5 * D]
    skip = pre[:, 5 * D:6 * D]

    s1, d1 = edge_index_rel[0], edge_index_rel[1]
    xls = jnp.take(xl, s1, axis=0, mode="clip")
    e = xls + jnp.take(xr, d1, axis=0, mode="clip")
    e = jnp.maximum(e, 0.2 * e)
    logits = jnp.einsum("ehc,hc->eh", e.reshape(-1, H, C), gat_att)
    ex = jnp.exp(logits)
    upd1 = jnp.concatenate(
        [(xls.reshape(-1, H, C) * ex[:, :, None]).reshape(-1, D), ex], axis=1)
    acc1 = jnp.zeros((N_BALL, D + H), jnp.float32).at[d1].add(upd1)
    num_rel = acc1[:, :D]
    rec_rel = 1.0 / (acc1[:, D:] + 1e-16)

    s2, d2 = edge_index_prec[0], edge_index_prec[1]
    ee = edge_attr_prec * tr_we
    kj = jnp.take(k, s2, axis=0, mode="clip") + ee
    vj = jnp.take(v, s2, axis=0, mode="clip") + ee
    qd = jnp.take(q, d2, axis=0, mode="clip")
    lg = jnp.sum((qd * kj).reshape(-1, H, C), axis=-1) * (1.0 / (C ** 0.5))
    ex2 = jnp.exp(lg)
    upd2 = jnp.concatenate(
        [(vj.reshape(-1, H, C) * ex2[:, :, None]).reshape(-1, D), ex2], axis=1)
    acc2 = jnp.zeros((N_BALL, D + H), jnp.float32).at[d2].add(upd2)
    num_prec = acc2[:, :D]
    rec_prec = 1.0 / (acc2[:, D:] + 1e-16)

    s3, d3 = edge_index_inf[0], edge_index_inf[1]
    feat3 = _sage_sc(x_ball, s3, d3)
    cnt3 = jnp.zeros((N_CTX, 1), jnp.float32).at[d3, 0].add(1.0)
    mean = feat3 / jnp.maximum(cnt3, 1.0)

    h_ball = _post_ball(num_rel, rec_rel, num_prec, rec_prec, skip, x_ball,
                        gat_bias, ln_ball_w, ln_ball_b, 2000)
    h_ctx = _post_ctx(mean, x_ctx, sage_wl, sage_wr, sage_bl, ln_ctx_w,
                      ln_ctx_b, 2000)
    return (h_ball, h_ctx)


# speedup vs baseline: 1.0086x; 1.0086x over previous
import functools

import jax
import jax.numpy as jnp
from jax import lax
from jax.experimental import pallas as pl
from jax.experimental.pallas import tpu as pltpu
from jax.experimental.pallas import tpu_sc as plsc

N_BALL = 50000
N_CTX = 10000
D = 128
H = 4
C = D // H

_NC = 2
_NS = 16
_NW = _NC * _NS

_SG_BLK = 128
_SG_NBLK = 25
_SG_TROW = 32
_SG_EPAD = _NW * _SG_NBLK * _SG_BLK
_SG_COLS = 128
_SG_ROWS = 10016
_SG_NCHUNK = _SG_ROWS // 32
_SG_CPT = (_SG_NCHUNK + _NS - 1) // _NS


def _sage_sc_body(x_hbm, sidx_hbm, didx_hbm, out_hbm,
                  acc, sidx_v, didx_v, rows_v, sem):
    c = lax.axis_index("c")
    s = lax.axis_index("s")
    tile = c * _NS + s

    zrow = jnp.zeros((16,), jnp.float32)

    def zrow_body(j, _):
        for l in range(8):
            rows_v[j, pl.ds(l * 16, 16)] = zrow
        return 0

    lax.fori_loop(0, _SG_BLK, zrow_body, 0)

    for r in range(_SG_CPT):
        chunk = s * _SG_CPT + r

        @pl.when(chunk < _SG_NCHUNK)
        def _zero_chunk(chunk=chunk):
            pltpu.sync_copy(rows_v.at[pl.ds(0, 32)],
                            acc.at[pl.ds(chunk * 32, 32)])

    plsc.subcore_barrier()

    pltpu.sync_copy(sidx_hbm.at[pl.ds(tile * _SG_TROW, _SG_TROW)], sidx_v)
    pltpu.sync_copy(didx_hbm.at[pl.ds(tile * _SG_TROW, _SG_TROW)], didx_v)

    def blk_body(j, _):
        pltpu.async_copy(x_hbm.at[sidx_v.at[j]], rows_v, sem).wait()
        pltpu.sync_copy(rows_v, acc.at[didx_v.at[j]], add=True)
        return 0

    lax.fori_loop(0, _SG_NBLK, blk_body, 0)
    plsc.subcore_barrier()

    for r in range(_SG_CPT):
        chunk = s * _SG_CPT + r

        @pl.when(chunk < _SG_NCHUNK)
        def _flush_chunk(chunk=chunk):
            pltpu.sync_copy(acc.at[pl.ds(chunk * 32, 32)],
                            rows_v.at[pl.ds(0, 32)])
            pltpu.sync_copy(rows_v.at[pl.ds(0, 32)],
                            out_hbm.at[pl.ds(c * _SG_ROWS + chunk * 32, 32)])


def _sage_sc(xaug, s3, d3):
    e = s3.shape[0]
    pad = _SG_EPAD - e
    ar = lax.iota(jnp.int32, pad)
    trash = _SG_ROWS - N_CTX
    s3p = jnp.concatenate([s3, (ar * 64) % N_BALL]
                          ).reshape(_NW, _SG_NBLK, _SG_BLK)
    d3p = jnp.concatenate([d3, N_CTX + (ar % trash)]
                          ).reshape(_NW, _SG_NBLK, _SG_BLK)
    zpad = jnp.zeros((_NW, _SG_TROW - _SG_NBLK, _SG_BLK), jnp.int32)
    s3p = jnp.concatenate([s3p, zpad], axis=1).reshape(-1, _SG_BLK)
    d3p = jnp.concatenate([d3p, zpad], axis=1).reshape(-1, _SG_BLK)
    kfn = pl.kernel(
        _sage_sc_body,
        out_type=jax.ShapeDtypeStruct((_NC * _SG_ROWS, _SG_COLS),
                                      jnp.float32),
        mesh=plsc.VectorSubcoreMesh(core_axis_name="c", subcore_axis_name="s"),
        scratch_types=[
            pltpu.VMEM_SHARED((_SG_ROWS, _SG_COLS), jnp.float32),
            pltpu.VMEM((_SG_TROW, _SG_BLK), jnp.int32),
            pltpu.VMEM((_SG_TROW, _SG_BLK), jnp.int32),
            pltpu.VMEM((_SG_BLK, _SG_COLS), jnp.float32),
            pltpu.SemaphoreType.DMA,
        ],
    )
    out = kfn(xaug, s3p, d3p)
    feat = out[:N_CTX] + out[_SG_ROWS:_SG_ROWS + N_CTX]
    return feat


def _pre_kernel(x_ref, w_ref, b_ref, o_ref):
    o_ref[...] = (
        jnp.dot(x_ref[...], w_ref[...], preferred_element_type=jnp.float32)
        + b_ref[...]
    )


def _fused_matmul(x, w, b, block):
    n, kdim = x.shape
    m = w.shape[1]
    grid = n // block
    return pl.pallas_call(
        _pre_kernel,
        grid=(grid,),
        in_specs=[
            pl.BlockSpec((block, kdim), lambda i: (i, 0)),
            pl.BlockSpec((kdim, m), lambda i: (0, 0)),
            pl.BlockSpec((1, m), lambda i: (0, 0)),
        ],
        out_specs=pl.BlockSpec((block, m), lambda i: (i, 0)),
        out_shape=jax.ShapeDtypeStruct((n, m), jnp.float32),
    )(x, w, b)


def _post_ball_kernel(numr_ref, rr_ref, nump_ref, rp_ref, skip_ref, x_ref,
                      gb_ref, eh_ref, lnw_ref, lnb_ref, o_ref):
    eh = eh_ref[...]
    t = (numr_ref[...] * jnp.dot(rr_ref[...], eh, preferred_element_type=jnp.float32)
         + nump_ref[...] * jnp.dot(rp_ref[...], eh, preferred_element_type=jnp.float32)
         + skip_ref[...] + x_ref[...] + gb_ref[...])
    mu = jnp.mean(t, axis=-1, keepdims=True)
    d = t - mu
    var = jnp.mean(d * d, axis=-1, keepdims=True)
    o_ref[...] = d * jax.lax.rsqrt(var + 1e-5) * lnw_ref[...] + lnb_ref[...]


def _post_ball(num_rel, rec_rel, num_prec, rec_prec, skip, x, gat_bias, lnw, lnb,
               block):
    n = x.shape[0]
    eh = jnp.repeat(jnp.eye(H, dtype=jnp.float32), C, axis=1)
    grid = n // block
    return pl.pallas_call(
        _post_ball_kernel,
        grid=(grid,),
        in_specs=[
            pl.BlockSpec((block, D), lambda i: (i, 0)),
            pl.BlockSpec((block, H), lambda i: (i, 0)),
            pl.BlockSpec((block, D), lambda i: (i, 0)),
            pl.BlockSpec((block, H), lambda i: (i, 0)),
            pl.BlockSpec((block, D), lambda i: (i, 0)),
            pl.BlockSpec((block, D), lambda i: (i, 0)),
            pl.BlockSpec((1, D), lambda i: (0, 0)),
            pl.BlockSpec((H, D), lambda i: (0, 0)),
            pl.BlockSpec((1, D), lambda i: (0, 0)),
            pl.BlockSpec((1, D), lambda i: (0, 0)),
        ],
        out_specs=pl.BlockSpec((block, D), lambda i: (i, 0)),
        out_shape=jax.ShapeDtypeStruct((n, D), jnp.float32),
    )(num_rel, rec_rel, num_prec, rec_prec, skip, x,
      gat_bias.reshape(1, D), eh, lnw.reshape(1, D), lnb.reshape(1, D))


def _post_ctx_kernel(mean_ref, x_ref, wl_ref, wr_ref, bl_ref, lnw_ref, lnb_ref,
                     o_ref):
    t = (jnp.dot(mean_ref[...], wl_ref[...], preferred_element_type=jnp.float32)
         + jnp.dot(x_ref[...], wr_ref[...], preferred_element_type=jnp.float32)
         + bl_ref[...] + x_ref[...])
    mu = jnp.mean(t, axis=-1, keepdims=True)
    d = t - mu
    var = jnp.mean(d * d, axis=-1, keepdims=True)
    o_ref[...] = d * jax.lax.rsqrt(var + 1e-5) * lnw_ref[...] + lnb_ref[...]


def _post_ctx(mean, x_ctx, wl, wr, bl, lnw, lnb, block):
    n = x_ctx.shape[0]
    grid = n // block
    return pl.pallas_call(
        _post_ctx_kernel,
        grid=(grid,),
        in_specs=[
            pl.BlockSpec((block, D), lambda i: (i, 0)),
            pl.BlockSpec((block, D), lambda i: (i, 0)),
            pl.BlockSpec((D, D), lambda i: (0, 0)),
            pl.BlockSpec((D, D), lambda i: (0, 0)),
            pl.BlockSpec((1, D), lambda i: (0, 0)),
            pl.BlockSpec((1, D), lambda i: (0, 0)),
            pl.BlockSpec((1, D), lambda i: (0, 0)),
        ],
        out_specs=pl.BlockSpec((block, D), lambda i: (i, 0)),
        out_shape=jax.ShapeDtypeStruct((n, D), jnp.float32),
    )(mean, x_ctx, wl, wr, bl.reshape(1, D), lnw.reshape(1, D),
      lnb.reshape(1, D))


def kernel(x_ball, x_ctx, edge_attr_prec, gat_wl, gat_bl, gat_wr, gat_br,
           gat_att, gat_bias, tr_wq, tr_bq, tr_wk, tr_bk, tr_wv, tr_bv, tr_we,
           tr_wskip, tr_bskip, sage_wl, sage_bl, sage_wr, ln_ball_w, ln_ball_b,
           ln_ctx_w, ln_ctx_b, edge_index_rel, edge_index_prec, edge_index_inf):
    w6 = jnp.concatenate([gat_wl, gat_wr, tr_wq, tr_wk, tr_wv, tr_wskip], axis=1)
    b6 = jnp.concatenate([gat_bl, gat_br, tr_bq, tr_bk, tr_bv, tr_bskip])
    pre = _fused_matmul(x_ball, w6, b6.reshape(1, 6 * D), 2000)
    xl = pre[:, 0 * D:1 * D]
    xr = pre[:, 1 * D:2 * D]
    q = pre[:, 2 * D:3 * D]
    k = pre[:, 3 * D:4 * D]
    v = pre[:, 4 * D:5 * D]
    skip = pre[:, 5 * D:6 * D]

    s1, d1 = edge_index_rel[0], edge_index_rel[1]
    xls = jnp.take(xl, s1, axis=0, mode="clip")
    e = xls + jnp.take(xr, d1, axis=0, mode="clip")
    e = jnp.maximum(e, 0.2 * e)
    logits = jnp.einsum("ehc,hc->eh", e.reshape(-1, H, C), gat_att)
    ex = jnp.exp(logits)
    num_rel = jax.ops.segment_sum(
        (xls.reshape(-1, H, C) * ex[:, :, None]).reshape(-1, D), d1,
        num_segments=N_BALL)
    s_rel = jax.ops.segment_sum(ex, d1, num_segments=N_BALL)
    rec_rel = 1.0 / (s_rel + 1e-16)

    s2, d2 = edge_index_prec[0], edge_index_prec[1]
    ee = edge_attr_prec * tr_we
    kj = jnp.take(k, s2, axis=0, mode="clip") + ee
    vj = jnp.take(v, s2, axis=0, mode="clip") + ee
    qd = jnp.take(q, d2, axis=0, mode="clip")
    lg = jnp.sum((qd * kj).reshape(-1, H, C), axis=-1) * (1.0 / (C ** 0.5))
    ex2 = jnp.exp(lg)
    num_prec = jax.ops.segment_sum(
        (vj.reshape(-1, H, C) * ex2[:, :, None]).reshape(-1, D), d2,
        num_segments=N_BALL)
    s_prec = jax.ops.segment_sum(ex2, d2, num_segments=N_BALL)
    rec_prec = 1.0 / (s_prec + 1e-16)

    s3, d3 = edge_index_inf[0], edge_index_inf[1]
    feat3 = _sage_sc(x_ball, s3, d3)
    cnt3 = jnp.zeros((N_CTX, 1), jnp.float32).at[d3, 0].add(1.0)
    mean = feat3 / jnp.maximum(cnt3, 1.0)

    h_ball = _post_ball(num_rel, rec_rel, num_prec, rec_prec, skip, x_ball,
                        gat_bias, ln_ball_w, ln_ball_b, 2000)
    h_ctx = _post_ctx(mean, x_ctx, sage_wl, sage_wr, sage_bl, ln_ctx_w,
                      ln_ctx_b, 2000)
    return (h_ball, h_ctx)
